# Initial kernel scaffold; baseline (speedup 1.0000x reference)
import jax
import jax.numpy as jnp
from jax.experimental import pallas as pl


def kernel(country_logits, region_logits, geocell_logits, pred_unit_xyz, uncertainty, embedding, positive_embedding, local_offset, base_centroid, geocell_probs, unit_xyz, geocell_centroids, country_id, region_id, geocell_id, geocell_to_country, geocell_to_region):
    raise NotImplementedError("write your pallas kernel here")


# speedup vs baseline: 3.6848x; 3.6848x over previous
import functools
import math

import jax
import jax.numpy as jnp
from jax.experimental import pallas as pl
from jax.experimental.pallas import tpu as pltpu

LS = 0.1
RAD = 100.0
TOPK = 32
TEMP = 0.07
ER = 6371.0088
WC = 0.3
WR = 0.3
WG = 1.0
WD = 0.5
WE = 0.2
WO = 1.0
WH = 0.1

_ACOS_C = (1.5707963050, -0.2145988016, 0.0889789874, -0.0501743046,
           0.0308918810, -0.0170881256, 0.0066700901, -0.0012624911)


def _acos(x):
    ax = jnp.abs(x)
    p = _ACOS_C[7]
    for c in (_ACOS_C[6], _ACOS_C[5], _ACOS_C[4], _ACOS_C[3], _ACOS_C[2],
              _ACOS_C[1], _ACOS_C[0]):
        p = p * ax + c
    r = jnp.sqrt(jnp.maximum(1.0 - ax, 0.0)) * p
    return jnp.where(x < 0, math.pi - r, r)


def _loss_kernel(B, G, CC, CR, D, BR,
                 cl_ref, rl_ref, gl_ref, gp_ref, emb_ref, pT_ref,
                 u_ref, pu_ref, lo_ref, bc_ref, unc_ref, cT_ref,
                 cid_ref, rid_ref, gid_ref, g2c_ref, g2r_ref,
                 out_ref,
                 work_ref, wd_ref, cmax_ref, csum_ref, acc_ref):
    i = pl.program_id(0)
    nb = pl.num_programs(0)

    @pl.when(i == 0)
    def _init():
        for k in range(8):
            acc_ref[k] = 0.0
        cmax_ref[...] = jnp.full_like(cmax_ref, -3e38)
        csum_ref[...] = jnp.zeros_like(csum_ref)

    f32 = jnp.float32

    cl = cl_ref[...]
    cm = jnp.max(cl, axis=1, keepdims=True)
    lse_c = jnp.log(jnp.sum(jnp.exp(cl - cm), axis=1, keepdims=True)) + cm
    cc_iota = jax.lax.broadcasted_iota(jnp.int32, (BR, CC), 1)
    cl_lbl = jnp.sum(jnp.where(cc_iota == cid_ref[...], cl, 0.0), axis=1,
                     keepdims=True)
    ce_c = lse_c - (1.0 - LS) * cl_lbl - LS * jnp.mean(cl, axis=1,
                                                       keepdims=True)

    rl = rl_ref[...]
    rm = jnp.max(rl, axis=1, keepdims=True)
    lse_r = jnp.log(jnp.sum(jnp.exp(rl - rm), axis=1, keepdims=True)) + rm
    cr_iota = jax.lax.broadcasted_iota(jnp.int32, (BR, CR), 1)
    rl_lbl = jnp.sum(jnp.where(cr_iota == rid_ref[...], rl, 0.0), axis=1,
                     keepdims=True)
    ce_r = lse_r - (1.0 - LS) * rl_lbl - LS * jnp.mean(rl, axis=1,
                                                       keepdims=True)

    c0 = cT_ref[0:1, :]
    c1 = cT_ref[1:2, :]
    c2 = cT_ref[2:3, :]
    inv = jax.lax.rsqrt(jnp.maximum(c0 * c0 + c1 * c1 + c2 * c2, 1e-24))
    u = u_ref[...]
    u0 = u[:, 0:1]
    u1 = u[:, 1:2]
    u2 = u[:, 2:3]
    cos = (u0 * (c0 * inv) + u1 * (c1 * inv) + u2 * (c2 * inv))
    cc = jnp.clip(cos, -1.0 + 1e-6, 1.0 - 1e-6)
    work_ref[...] = cc
    wd_ref[...] = jnp.zeros_like(wd_ref)
    lane = jax.lax.broadcasted_iota(jnp.int32, (BR, G), 1)

    def _body(t, carry):
        w = work_ref[...]
        m = jnp.max(w, axis=1, keepdims=True)
        idx = jnp.min(jnp.where(w == m, lane, G), axis=1, keepdims=True)
        sel = lane == idx
        d = _acos(m) * ER
        nw = jnp.exp(-(d / RAD) ** 2)
        wd_ref[...] = jnp.where(sel, nw, wd_ref[...])
        work_ref[...] = jnp.where(sel, -2.0, w)
        return carry

    jax.lax.fori_loop(0, TOPK, _body, 0)

    gl = gl_ref[...]
    gm = jnp.max(gl, axis=1, keepdims=True)
    lse_g = jnp.log(jnp.sum(jnp.exp(gl - gm), axis=1, keepdims=True)) + gm
    wdense = wd_ref[...] + jnp.where(lane == gid_ref[...], 1.0, 0.0)
    ssum = jnp.sum(wdense, axis=1, keepdims=True)
    wlog = jnp.sum(wdense * gl, axis=1, keepdims=True)
    geo_row = lse_g - wlog / jnp.maximum(ssum, 1e-6)

    pu = pu_ref[...]
    dot = jnp.sum(pu * u, axis=1, keepdims=True)
    gcd = _acos(jnp.clip(dot, -1.0 + 1e-6, 1.0 - 1e-6)) * ER
    scale = unc_ref[...]
    geod_row = gcd / scale + jnp.log(scale)

    bc = bc_ref[...]
    tb = jnp.sum(u * bc, axis=1, keepdims=True)
    dv = lo_ref[...] - (u - tb * bc)
    ad = jnp.abs(dv)
    off_blk = jnp.sum(jnp.where(ad < 1.0, 0.5 * dv * dv, ad - 0.5))

    clog = jax.lax.dot_general(
        emb_ref[...], pT_ref[...], (((1,), (0,)), ((), ())),
        preferred_element_type=f32,
        precision=jax.lax.Precision.HIGHEST) * (1.0 / TEMP)
    km = jnp.max(clog, axis=1, keepdims=True)
    lse_row = jnp.log(jnp.sum(jnp.exp(clog - km), axis=1, keepdims=True)) + km
    row_i = jax.lax.broadcasted_iota(jnp.int32, (BR, B), 0) + i * BR
    col_i = jax.lax.broadcasted_iota(jnp.int32, (BR, B), 1)
    diag = jnp.sum(jnp.where(row_i == col_i, clog, 0.0), axis=1,
                   keepdims=True)
    prev_m = cmax_ref[...]
    bm = jnp.max(clog, axis=0, keepdims=True)
    new_m = jnp.maximum(prev_m, bm)
    csum_ref[...] = (csum_ref[...] * jnp.exp(prev_m - new_m)
                     + jnp.sum(jnp.exp(clog - new_m), axis=0, keepdims=True))
    cmax_ref[...] = new_m

    cc_g = jax.lax.broadcasted_iota(jnp.int32, (G, CC), 1)
    memc = jnp.where(cc_g == g2c_ref[...], 1.0, 0.0)
    cr_g = jax.lax.broadcasted_iota(jnp.int32, (G, CR), 1)
    memr = jnp.where(cr_g == g2r_ref[...], 1.0, 0.0)
    gp = gp_ref[...]
    tc = jnp.maximum(jax.lax.dot_general(
        gp, memc, (((1,), (0,)), ((), ())), preferred_element_type=f32,
        precision=jax.lax.Precision.HIGHEST), 1e-6)
    tc = tc / jnp.sum(tc, axis=1, keepdims=True)
    tr = jnp.maximum(jax.lax.dot_general(
        gp, memr, (((1,), (0,)), ((), ())), preferred_element_type=f32,
        precision=jax.lax.Precision.HIGHEST), 1e-6)
    tr = tr / jnp.sum(tr, axis=1, keepdims=True)
    klc_row = (jnp.sum(tc * jnp.log(tc), axis=1, keepdims=True)
               - jnp.sum(tc * cl, axis=1, keepdims=True) + lse_c)
    klr_row = (jnp.sum(tr * jnp.log(tr), axis=1, keepdims=True)
               - jnp.sum(tr * rl, axis=1, keepdims=True) + lse_r)

    acc_ref[0] += jnp.sum(ce_c)
    acc_ref[1] += jnp.sum(ce_r)
    acc_ref[2] += jnp.sum(geo_row)
    acc_ref[3] += jnp.sum(geod_row)
    acc_ref[4] += off_blk
    acc_ref[5] += jnp.sum(klc_row + klr_row)
    acc_ref[6] += jnp.sum(lse_row)
    acc_ref[7] += jnp.sum(diag)

    @pl.when(i == nb - 1)
    def _fin():
        col_lse = jnp.log(csum_ref[...]) + cmax_ref[...]
        col_sum = jnp.sum(col_lse)
        fb = f32(B)
        country = acc_ref[0] / fb
        region = acc_ref[1] / fb
        geocell = acc_ref[2] / fb
        geodesic = acc_ref[3] / fb
        offset = acc_ref[4] / (3.0 * fb)
        hierarchy = acc_ref[5] / fb
        embedding = (acc_ref[6] + col_sum - 2.0 * acc_ref[7]) / (2.0 * fb)
        total = (WC * country + WR * region + WG * geocell + WD * geodesic
                 + WE * embedding + WO * offset + WH * hierarchy)
        out_ref[0] = total
        out_ref[1] = country
        out_ref[2] = region
        out_ref[3] = geocell
        out_ref[4] = geodesic
        out_ref[5] = embedding
        out_ref[6] = offset
        out_ref[7] = hierarchy


def _run(country_logits, region_logits, geocell_logits, geocell_probs,
         embedding, pos_T, unit_xyz, pred_unit_xyz, local_offset,
         base_centroid, uncertainty, cent_T, cid, rid, gid, g2c, g2r,
         BR, interpret=False):
    B, CC = country_logits.shape
    CR = region_logits.shape[1]
    G = geocell_logits.shape[1]
    D = embedding.shape[1]
    nb = B // BR

    def blk(d):
        return pl.BlockSpec((BR, d), lambda i: (i, 0))

    def full(s):
        return pl.BlockSpec(s, lambda i: tuple(0 for _ in s))

    out = pl.pallas_call(
        functools.partial(_loss_kernel, B, G, CC, CR, D, BR),
        grid=(nb,),
        in_specs=[
            blk(CC), blk(CR), blk(G), blk(G), blk(D), full((D, B)),
            blk(3), blk(3), blk(3), blk(3), blk(1), full((3, G)),
            blk(1), blk(1), blk(1), full((G, 1)), full((G, 1)),
        ],
        out_specs=pl.BlockSpec(memory_space=pltpu.SMEM),
        out_shape=jax.ShapeDtypeStruct((8,), jnp.float32),
        scratch_shapes=[
            pltpu.VMEM((BR, G), jnp.float32),
            pltpu.VMEM((BR, G), jnp.float32),
            pltpu.VMEM((1, B), jnp.float32),
            pltpu.VMEM((1, B), jnp.float32),
            pltpu.SMEM((8,), jnp.float32),
        ],
        interpret=interpret,
    )(country_logits, region_logits, geocell_logits, geocell_probs,
      embedding, pos_T, unit_xyz, pred_unit_xyz, local_offset,
      base_centroid, uncertainty, cent_T, cid, rid, gid, g2c, g2r)
    return out


def kernel(country_logits, region_logits, geocell_logits, pred_unit_xyz,
           uncertainty, embedding, positive_embedding, local_offset,
           base_centroid, geocell_probs, unit_xyz, geocell_centroids,
           country_id, region_id, geocell_id, geocell_to_country,
           geocell_to_region):
    B = country_logits.shape[0]
    out = _run(
        country_logits, region_logits, geocell_logits, geocell_probs,
        embedding, positive_embedding.T, unit_xyz, pred_unit_xyz,
        local_offset, base_centroid, uncertainty, geocell_centroids.T,
        country_id.astype(jnp.int32).reshape(B, 1),
        region_id.astype(jnp.int32).reshape(B, 1),
        geocell_id.astype(jnp.int32).reshape(B, 1),
        geocell_to_country.astype(jnp.int32).reshape(-1, 1),
        geocell_to_region.astype(jnp.int32).reshape(-1, 1),
        BR=B // 16)
    return tuple(out[k] for k in range(8))


# speedup vs baseline: 7.3477x; 1.9940x over previous
import functools
import math
import struct

import jax
import jax.numpy as jnp
from jax.experimental import pallas as pl
from jax.experimental.pallas import tpu as pltpu

LS = 0.1
RAD = 100.0
TOPK = 32
TEMP = 0.07
ER = 6371.0088
WC = 0.3
WR = 0.3
WG = 1.0
WD = 0.5
WE = 0.2
WO = 1.0
WH = 0.1

NW_T = 0.9966


def _f32_bits(x):
    return struct.unpack('<i', struct.pack('<f', x))[0]


_TBITS = _f32_bits(NW_T)
_HIBITS = _f32_bits(1.0 - 1e-6)
_VITERS = (_HIBITS - (_TBITS - 1)).bit_length()

_ACOS_C = (1.5707963050, -0.2145988016, 0.0889789874, -0.0501743046,
           0.0308918810, -0.0170881256, 0.0066700901, -0.0012624911)


def _acos(x):
    ax = jnp.abs(x)
    p = _ACOS_C[7]
    for c in (_ACOS_C[6], _ACOS_C[5], _ACOS_C[4], _ACOS_C[3], _ACOS_C[2],
              _ACOS_C[1], _ACOS_C[0]):
        p = p * ax + c
    r = jnp.sqrt(jnp.maximum(1.0 - ax, 0.0)) * p
    return jnp.where(x < 0, math.pi - r, r)


def _loss_kernel(B, G, CC, CR, D, BR,
                 cl_ref, rl_ref, gl_ref, gp_ref, emb_ref, pT_ref, pblk_ref,
                 u_ref, pu_ref, lo_ref, bc_ref, unc_ref, cT_ref,
                 cid_ref, rid_ref, gid_ref, g2c_ref, g2r_ref,
                 out_ref,
                 work_ref, memc_ref, memr_ref, cmax_ref, csum_ref, acc_ref):
    i = pl.program_id(0)
    nb = pl.num_programs(0)

    @pl.when(i == 0)
    def _init():
        for k in range(8):
            acc_ref[k] = 0.0
        cmax_ref[...] = jnp.full_like(cmax_ref, -3e38)
        csum_ref[...] = jnp.zeros_like(csum_ref)

    f32 = jnp.float32

    cl = cl_ref[...]
    cm = jnp.max(cl, axis=1, keepdims=True)
    lse_c = jnp.log(jnp.sum(jnp.exp(cl - cm), axis=1, keepdims=True)) + cm
    cc_iota = jax.lax.broadcasted_iota(jnp.int32, (BR, CC), 1)
    cl_lbl = jnp.sum(jnp.where(cc_iota == cid_ref[...], cl, 0.0), axis=1,
                     keepdims=True)
    ce_c = lse_c - (1.0 - LS) * cl_lbl - LS * jnp.mean(cl, axis=1,
                                                       keepdims=True)

    rl = rl_ref[...]
    rm = jnp.max(rl, axis=1, keepdims=True)
    lse_r = jnp.log(jnp.sum(jnp.exp(rl - rm), axis=1, keepdims=True)) + rm
    cr_iota = jax.lax.broadcasted_iota(jnp.int32, (BR, CR), 1)
    rl_lbl = jnp.sum(jnp.where(cr_iota == rid_ref[...], rl, 0.0), axis=1,
                     keepdims=True)
    ce_r = lse_r - (1.0 - LS) * rl_lbl - LS * jnp.mean(rl, axis=1,
                                                       keepdims=True)

    c0 = cT_ref[0:1, :]
    c1 = cT_ref[1:2, :]
    c2 = cT_ref[2:3, :]
    inv = jax.lax.rsqrt(jnp.maximum(c0 * c0 + c1 * c1 + c2 * c2, 1e-24))
    u = u_ref[...]
    u0 = u[:, 0:1]
    u1 = u[:, 1:2]
    u2 = u[:, 2:3]
    cos = (u0 * (c0 * inv) + u1 * (c1 * inv) + u2 * (c2 * inv))
    cc = jnp.clip(cos, -1.0 + 1e-6, 1.0 - 1e-6)
    work_ref[...] = cc
    lane = jax.lax.broadcasted_iota(jnp.int32, (BR, G), 1)
    i32 = jnp.int32
    fTOPK = f32(TOPK)

    def _vbody(t, carry):
        lo, hi = carry
        mid = jax.lax.shift_right_logical(lo + hi + 1, 1)
        bits = jax.lax.bitcast_convert_type(work_ref[...], i32)
        cnt = jnp.sum(jnp.where(bits >= mid, 1.0, 0.0), axis=1,
                      keepdims=True)
        ok = cnt >= fTOPK
        return (jnp.where(ok, mid, lo), jnp.where(ok, hi, mid - 1))

    vlo, _ = jax.lax.fori_loop(
        0, _VITERS, _vbody,
        (jnp.full((BR, 1), _TBITS - 1, i32),
         jnp.full((BR, 1), _HIBITS, i32)))
    bits = jax.lax.bitcast_convert_type(work_ref[...], i32)
    gt = bits > vlo
    eq = bits == vlo
    ngt = jnp.sum(jnp.where(gt, 1.0, 0.0), axis=1, keepdims=True)
    mtie = fTOPK - ngt

    def _ibody(t, carry):
        glo, ghi = carry
        mid = jax.lax.shift_right_logical(glo + ghi, 1)
        bb = jax.lax.bitcast_convert_type(work_ref[...], i32)
        c = jnp.sum(jnp.where((bb == vlo) & (lane <= mid), 1.0, 0.0),
                    axis=1, keepdims=True)
        ok = c >= mtie
        return (jnp.where(ok, glo, mid + 1), jnp.where(ok, mid, ghi))

    _, gstar = jax.lax.fori_loop(
        0, max(1, (G - 1).bit_length()), _ibody,
        (jnp.zeros((BR, 1), i32), jnp.full((BR, 1), G - 1, i32)))

    sel = gt | (eq & (lane <= gstar))
    nw = jnp.exp(-(_acos(cc) * (ER / RAD)) ** 2)
    w = jnp.where(sel, nw, 0.0)
    ssum = jnp.sum(w, axis=1, keepdims=True) + 1.0

    gl = gl_ref[...]
    gm = jnp.max(gl, axis=1, keepdims=True)
    lse_g = jnp.log(jnp.sum(jnp.exp(gl - gm), axis=1, keepdims=True)) + gm
    wlog = (jnp.sum(w * gl, axis=1, keepdims=True)
            + jnp.sum(jnp.where(lane == gid_ref[...], gl, 0.0), axis=1,
                      keepdims=True))
    geo_row = lse_g - wlog / ssum

    pu = pu_ref[...]
    dot = jnp.sum(pu * u, axis=1, keepdims=True)
    gcd = _acos(jnp.clip(dot, -1.0 + 1e-6, 1.0 - 1e-6)) * ER
    scale = unc_ref[...]
    geod_row = gcd / scale + jnp.log(scale)

    bc = bc_ref[...]
    tb = jnp.sum(u * bc, axis=1, keepdims=True)
    dv = lo_ref[...] - (u - tb * bc)
    ad = jnp.abs(dv)
    off_blk = jnp.sum(jnp.where(ad < 1.0, 0.5 * dv * dv, ad - 0.5))

    clog = jax.lax.dot_general(
        emb_ref[...], pT_ref[...], (((1,), (0,)), ((), ())),
        preferred_element_type=f32,
        precision=jax.lax.Precision.DEFAULT) * (1.0 / TEMP)
    km = jnp.max(clog, axis=1, keepdims=True)
    lse_row = jnp.log(jnp.sum(jnp.exp(clog - km), axis=1, keepdims=True)) + km
    diag = jnp.sum(emb_ref[...] * pblk_ref[...], axis=1,
                   keepdims=True) * (1.0 / TEMP)
    prev_m = cmax_ref[...]
    bm = jnp.max(clog, axis=0, keepdims=True)
    new_m = jnp.maximum(prev_m, bm)
    csum_ref[...] = (csum_ref[...] * jnp.exp(prev_m - new_m)
                     + jnp.sum(jnp.exp(clog - new_m), axis=0, keepdims=True))
    cmax_ref[...] = new_m

    @pl.when(i == 0)
    def _onehots():
        cc_g = jax.lax.broadcasted_iota(jnp.int32, (G, CC), 1)
        memc_ref[...] = jnp.where(cc_g == g2c_ref[...], 1.0, 0.0)
        cr_g = jax.lax.broadcasted_iota(jnp.int32, (G, CR), 1)
        memr_ref[...] = jnp.where(cr_g == g2r_ref[...], 1.0, 0.0)

    gp = gp_ref[...]
    tc = jnp.maximum(jax.lax.dot_general(
        gp, memc_ref[...], (((1,), (0,)), ((), ())),
        preferred_element_type=f32,
        precision=jax.lax.Precision.DEFAULT), 1e-6)
    tc = tc / jnp.sum(tc, axis=1, keepdims=True)
    tr = jnp.maximum(jax.lax.dot_general(
        gp, memr_ref[...], (((1,), (0,)), ((), ())),
        preferred_element_type=f32,
        precision=jax.lax.Precision.DEFAULT), 1e-6)
    tr = tr / jnp.sum(tr, axis=1, keepdims=True)
    klc_row = (jnp.sum(tc * jnp.log(tc), axis=1, keepdims=True)
               - jnp.sum(tc * cl, axis=1, keepdims=True) + lse_c)
    klr_row = (jnp.sum(tr * jnp.log(tr), axis=1, keepdims=True)
               - jnp.sum(tr * rl, axis=1, keepdims=True) + lse_r)

    acc_ref[0] += jnp.sum(ce_c)
    acc_ref[1] += jnp.sum(ce_r)
    acc_ref[2] += jnp.sum(geo_row)
    acc_ref[3] += jnp.sum(geod_row)
    acc_ref[4] += off_blk
    acc_ref[5] += jnp.sum(klc_row + klr_row)
    acc_ref[6] += jnp.sum(lse_row)
    acc_ref[7] += jnp.sum(diag)

    @pl.when(i == nb - 1)
    def _fin():
        col_lse = jnp.log(csum_ref[...]) + cmax_ref[...]
        col_sum = jnp.sum(col_lse)
        fb = f32(B)
        country = acc_ref[0] / fb
        region = acc_ref[1] / fb
        geocell = acc_ref[2] / fb
        geodesic = acc_ref[3] / fb
        offset = acc_ref[4] / (3.0 * fb)
        hierarchy = acc_ref[5] / fb
        embedding = (acc_ref[6] + col_sum - 2.0 * acc_ref[7]) / (2.0 * fb)
        total = (WC * country + WR * region + WG * geocell + WD * geodesic
                 + WE * embedding + WO * offset + WH * hierarchy)
        out_ref[0] = total
        out_ref[1] = country
        out_ref[2] = region
        out_ref[3] = geocell
        out_ref[4] = geodesic
        out_ref[5] = embedding
        out_ref[6] = offset
        out_ref[7] = hierarchy


def _run(country_logits, region_logits, geocell_logits, geocell_probs,
         embedding, pos_T, pos_blk, unit_xyz, pred_unit_xyz, local_offset,
         base_centroid, uncertainty, cent_T, cid, rid, gid, g2c, g2r,
         BR, interpret=False):
    B, CC = country_logits.shape
    CR = region_logits.shape[1]
    G = geocell_logits.shape[1]
    D = embedding.shape[1]
    nb = B // BR

    def blk(d):
        return pl.BlockSpec((BR, d), lambda i: (i, 0))

    def full(s):
        return pl.BlockSpec(s, lambda i: tuple(0 for _ in s))

    out = pl.pallas_call(
        functools.partial(_loss_kernel, B, G, CC, CR, D, BR),
        grid=(nb,),
        in_specs=[
            blk(CC), blk(CR), blk(G), blk(G), blk(D), full((D, B)), blk(D),
            blk(3), blk(3), blk(3), blk(3), blk(1), full((3, G)),
            blk(1), blk(1), blk(1), full((G, 1)), full((G, 1)),
        ],
        out_specs=pl.BlockSpec(memory_space=pltpu.SMEM),
        out_shape=jax.ShapeDtypeStruct((8,), jnp.float32),
        scratch_shapes=[
            pltpu.VMEM((BR, G), jnp.float32),
            pltpu.VMEM((G, CC), jnp.float32),
            pltpu.VMEM((G, CR), jnp.float32),
            pltpu.VMEM((1, B), jnp.float32),
            pltpu.VMEM((1, B), jnp.float32),
            pltpu.SMEM((8,), jnp.float32),
        ],
        interpret=interpret,
    )(country_logits, region_logits, geocell_logits, geocell_probs,
      embedding, pos_T, pos_blk, unit_xyz, pred_unit_xyz, local_offset,
      base_centroid, uncertainty, cent_T, cid, rid, gid, g2c, g2r)
    return out


def kernel(country_logits, region_logits, geocell_logits, pred_unit_xyz,
           uncertainty, embedding, positive_embedding, local_offset,
           base_centroid, geocell_probs, unit_xyz, geocell_centroids,
           country_id, region_id, geocell_id, geocell_to_country,
           geocell_to_region):
    B = country_logits.shape[0]
    out = _run(
        country_logits, region_logits, geocell_logits, geocell_probs,
        embedding, positive_embedding.T, positive_embedding,
        unit_xyz, pred_unit_xyz,
        local_offset, base_centroid, uncertainty, geocell_centroids.T,
        country_id.astype(jnp.int32).reshape(B, 1),
        region_id.astype(jnp.int32).reshape(B, 1),
        geocell_id.astype(jnp.int32).reshape(B, 1),
        geocell_to_country.astype(jnp.int32).reshape(-1, 1),
        geocell_to_region.astype(jnp.int32).reshape(-1, 1),
        BR=B // 16)
    return tuple(out[k] for k in range(8))


# speedup vs baseline: 9.1360x; 1.2434x over previous
import functools
import math
import struct

import jax
import jax.numpy as jnp
from jax.experimental import pallas as pl
from jax.experimental.pallas import tpu as pltpu

LS = 0.1
RAD = 100.0
TOPK = 32
TEMP = 0.07
ER = 6371.0088
WC = 0.3
WR = 0.3
WG = 1.0
WD = 0.5
WE = 0.2
WO = 1.0
WH = 0.1

NW_T = 0.9966


def _f32_bits(x):
    return struct.unpack('<i', struct.pack('<f', x))[0]


_TBITS = _f32_bits(NW_T)
_HIBITS = _f32_bits(1.0 - 1e-6)
_VITERS = (_HIBITS - (_TBITS - 1)).bit_length()
_UMAX = 1.0 - NW_T

_ACOS_C = (1.5707963050, -0.2145988016, 0.0889789874, -0.0501743046,
           0.0308918810, -0.0170881256, 0.0066700901, -0.0012624911)


def _acos(x):
    ax = jnp.abs(x)
    p = _ACOS_C[7]
    for c in (_ACOS_C[6], _ACOS_C[5], _ACOS_C[4], _ACOS_C[3], _ACOS_C[2],
              _ACOS_C[1], _ACOS_C[0]):
        p = p * ax + c
    r = jnp.sqrt(jnp.maximum(1.0 - ax, 0.0)) * p
    return jnp.where(x < 0, math.pi - r, r)


def _loss_kernel(B, G, CC, CR, D, BR,
                 cl_ref, rl_ref, gl_ref, gp_ref, emb_ref, pT_ref, pblk_ref,
                 u_ref, pu_ref, lo_ref, bc_ref, unc_ref, cT_ref,
                 cid_ref, rid_ref, gid_ref, g2c_ref, g2r_ref,
                 out_ref,
                 work_ref, memc_ref, memr_ref, tri_ref, cmax_ref, csum_ref,
                 acc_ref):
    i = pl.program_id(0)
    nb = pl.num_programs(0)

    @pl.when(i == 0)
    def _init():
        for k in range(8):
            acc_ref[k] = 0.0
        cmax_ref[...] = jnp.full_like(cmax_ref, -3e38)
        csum_ref[...] = jnp.zeros_like(csum_ref)
        ih = jax.lax.broadcasted_iota(jnp.int32, (G, G), 0)
        ig = jax.lax.broadcasted_iota(jnp.int32, (G, G), 1)
        tri_ref[...] = jnp.where(ih <= ig, 1.0, 0.0).astype(jnp.bfloat16)

    f32 = jnp.float32

    cl = cl_ref[...]
    cm = jnp.max(cl, axis=1, keepdims=True)
    lse_c = jnp.log(jnp.sum(jnp.exp(cl - cm), axis=1, keepdims=True)) + cm
    cc_iota = jax.lax.broadcasted_iota(jnp.int32, (BR, CC), 1)
    cl_lbl = jnp.sum(jnp.where(cc_iota == cid_ref[...], cl, 0.0), axis=1,
                     keepdims=True)
    ce_c = lse_c - (1.0 - LS) * cl_lbl - LS * jnp.mean(cl, axis=1,
                                                       keepdims=True)

    rl = rl_ref[...]
    rm = jnp.max(rl, axis=1, keepdims=True)
    lse_r = jnp.log(jnp.sum(jnp.exp(rl - rm), axis=1, keepdims=True)) + rm
    cr_iota = jax.lax.broadcasted_iota(jnp.int32, (BR, CR), 1)
    rl_lbl = jnp.sum(jnp.where(cr_iota == rid_ref[...], rl, 0.0), axis=1,
                     keepdims=True)
    ce_r = lse_r - (1.0 - LS) * rl_lbl - LS * jnp.mean(rl, axis=1,
                                                       keepdims=True)

    c0 = cT_ref[0:1, :]
    c1 = cT_ref[1:2, :]
    c2 = cT_ref[2:3, :]
    inv = jax.lax.rsqrt(jnp.maximum(c0 * c0 + c1 * c1 + c2 * c2, 1e-24))
    u = u_ref[...]
    u0 = u[:, 0:1]
    u1 = u[:, 1:2]
    u2 = u[:, 2:3]
    cos = (u0 * (c0 * inv) + u1 * (c1 * inv) + u2 * (c2 * inv))
    cc = jnp.clip(cos, -1.0 + 1e-6, 1.0 - 1e-6)
    work_ref[...] = cc
    lane = jax.lax.broadcasted_iota(jnp.int32, (BR, G), 1)
    i32 = jnp.int32
    fTOPK = f32(TOPK)

    def _vbody(t, carry):
        lo, hi = carry
        mid = jax.lax.shift_right_logical(lo + hi + 1, 1)
        bits = jax.lax.bitcast_convert_type(work_ref[...], i32)
        cnt = jnp.sum(jnp.where(bits >= mid, 1.0, 0.0), axis=1,
                      keepdims=True)
        ok = cnt >= fTOPK
        return (jnp.where(ok, mid, lo), jnp.where(ok, hi, mid - 1))

    vlo, _ = jax.lax.fori_loop(
        0, _VITERS, _vbody,
        (jnp.full((BR, 1), _TBITS - 1, i32),
         jnp.full((BR, 1), _HIBITS, i32)))
    bits = jax.lax.bitcast_convert_type(work_ref[...], i32)
    gt = bits > vlo
    eq = bits == vlo
    ngt = jnp.sum(jnp.where(gt, 1.0, 0.0), axis=1, keepdims=True)
    mtie = fTOPK - ngt

    eq_bf = jnp.where(eq, 1.0, 0.0).astype(jnp.bfloat16)
    prefix = jax.lax.dot_general(
        eq_bf, tri_ref[...], (((1,), (0,)), ((), ())),
        preferred_element_type=f32)
    sel = gt | (eq & (prefix <= mtie))

    u1m = 1.0 - cc
    expo = (2.0 * (ER / RAD) ** 2) * u1m * (
        1.0 + u1m * (1.0 / 6.0) + (u1m * u1m) * (2.0 / 45.0))
    nw = jnp.where(u1m <= _UMAX, jnp.exp(-expo), 0.0)
    w = jnp.where(sel, nw, 0.0)
    ssum = jnp.sum(w, axis=1, keepdims=True) + 1.0

    gl = gl_ref[...]
    gm = jnp.max(gl, axis=1, keepdims=True)
    lse_g = jnp.log(jnp.sum(jnp.exp(gl - gm), axis=1, keepdims=True)) + gm
    wlog = (jnp.sum(w * gl, axis=1, keepdims=True)
            + jnp.sum(jnp.where(lane == gid_ref[...], gl, 0.0), axis=1,
                      keepdims=True))
    geo_row = lse_g - wlog / ssum

    pu = pu_ref[...]
    dot = jnp.sum(pu * u, axis=1, keepdims=True)
    gcd = _acos(jnp.clip(dot, -1.0 + 1e-6, 1.0 - 1e-6)) * ER
    scale = unc_ref[...]
    geod_row = gcd / scale + jnp.log(scale)

    bc = bc_ref[...]
    tb = jnp.sum(u * bc, axis=1, keepdims=True)
    dv = lo_ref[...] - (u - tb * bc)
    ad = jnp.abs(dv)
    off_blk = jnp.sum(jnp.where(ad < 1.0, 0.5 * dv * dv, ad - 0.5))

    clog = jax.lax.dot_general(
        emb_ref[...], pT_ref[...], (((1,), (0,)), ((), ())),
        preferred_element_type=f32,
        precision=jax.lax.Precision.DEFAULT) * (1.0 / TEMP)
    km = jnp.max(clog, axis=1, keepdims=True)
    lse_row = jnp.log(jnp.sum(jnp.exp(clog - km), axis=1, keepdims=True)) + km
    diag = jnp.sum(emb_ref[...] * pblk_ref[...], axis=1,
                   keepdims=True) * (1.0 / TEMP)
    prev_m = cmax_ref[...]
    bm = jnp.max(clog, axis=0, keepdims=True)
    new_m = jnp.maximum(prev_m, bm)
    csum_ref[...] = (csum_ref[...] * jnp.exp(prev_m - new_m)
                     + jnp.sum(jnp.exp(clog - new_m), axis=0, keepdims=True))
    cmax_ref[...] = new_m

    @pl.when(i == 0)
    def _onehots():
        cc_g = jax.lax.broadcasted_iota(jnp.int32, (G, CC), 1)
        memc_ref[...] = jnp.where(cc_g == g2c_ref[...], 1.0, 0.0)
        cr_g = jax.lax.broadcasted_iota(jnp.int32, (G, CR), 1)
        memr_ref[...] = jnp.where(cr_g == g2r_ref[...], 1.0, 0.0)

    gp = gp_ref[...]
    tc = jnp.maximum(jax.lax.dot_general(
        gp, memc_ref[...], (((1,), (0,)), ((), ())),
        preferred_element_type=f32,
        precision=jax.lax.Precision.DEFAULT), 1e-6)
    tc = tc / jnp.sum(tc, axis=1, keepdims=True)
    tr = jnp.maximum(jax.lax.dot_general(
        gp, memr_ref[...], (((1,), (0,)), ((), ())),
        preferred_element_type=f32,
        precision=jax.lax.Precision.DEFAULT), 1e-6)
    tr = tr / jnp.sum(tr, axis=1, keepdims=True)
    klc_row = (jnp.sum(tc * jnp.log(tc), axis=1, keepdims=True)
               - jnp.sum(tc * cl, axis=1, keepdims=True) + lse_c)
    klr_row = (jnp.sum(tr * jnp.log(tr), axis=1, keepdims=True)
               - jnp.sum(tr * rl, axis=1, keepdims=True) + lse_r)

    acc_ref[0] += jnp.sum(ce_c)
    acc_ref[1] += jnp.sum(ce_r)
    acc_ref[2] += jnp.sum(geo_row)
    acc_ref[3] += jnp.sum(geod_row)
    acc_ref[4] += off_blk
    acc_ref[5] += jnp.sum(klc_row + klr_row)
    acc_ref[6] += jnp.sum(lse_row)
    acc_ref[7] += jnp.sum(diag)

    @pl.when(i == nb - 1)
    def _fin():
        col_lse = jnp.log(csum_ref[...]) + cmax_ref[...]
        col_sum = jnp.sum(col_lse)
        fb = f32(B)
        country = acc_ref[0] / fb
        region = acc_ref[1] / fb
        geocell = acc_ref[2] / fb
        geodesic = acc_ref[3] / fb
        offset = acc_ref[4] / (3.0 * fb)
        hierarchy = acc_ref[5] / fb
        embedding = (acc_ref[6] + col_sum - 2.0 * acc_ref[7]) / (2.0 * fb)
        total = (WC * country + WR * region + WG * geocell + WD * geodesic
                 + WE * embedding + WO * offset + WH * hierarchy)
        out_ref[0] = total
        out_ref[1] = country
        out_ref[2] = region
        out_ref[3] = geocell
        out_ref[4] = geodesic
        out_ref[5] = embedding
        out_ref[6] = offset
        out_ref[7] = hierarchy


def _run(country_logits, region_logits, geocell_logits, geocell_probs,
         embedding, pos_T, pos_blk, unit_xyz, pred_unit_xyz, local_offset,
         base_centroid, uncertainty, cent_T, cid, rid, gid, g2c, g2r,
         BR, interpret=False):
    B, CC = country_logits.shape
    CR = region_logits.shape[1]
    G = geocell_logits.shape[1]
    D = embedding.shape[1]
    nb = B // BR

    def blk(d):
        return pl.BlockSpec((BR, d), lambda i: (i, 0))

    def full(s):
        return pl.BlockSpec(s, lambda i: tuple(0 for _ in s))

    out = pl.pallas_call(
        functools.partial(_loss_kernel, B, G, CC, CR, D, BR),
        grid=(nb,),
        in_specs=[
            blk(CC), blk(CR), blk(G), blk(G), blk(D), full((D, B)), blk(D),
            blk(3), blk(3), blk(3), blk(3), blk(1), full((3, G)),
            blk(1), blk(1), blk(1), full((G, 1)), full((G, 1)),
        ],
        out_specs=pl.BlockSpec(memory_space=pltpu.SMEM),
        out_shape=jax.ShapeDtypeStruct((8,), jnp.float32),
        scratch_shapes=[
            pltpu.VMEM((BR, G), jnp.float32),
            pltpu.VMEM((G, CC), jnp.float32),
            pltpu.VMEM((G, CR), jnp.float32),
            pltpu.VMEM((G, G), jnp.bfloat16),
            pltpu.VMEM((1, B), jnp.float32),
            pltpu.VMEM((1, B), jnp.float32),
            pltpu.SMEM((8,), jnp.float32),
        ],
        interpret=interpret,
    )(country_logits, region_logits, geocell_logits, geocell_probs,
      embedding, pos_T, pos_blk, unit_xyz, pred_unit_xyz, local_offset,
      base_centroid, uncertainty, cent_T, cid, rid, gid, g2c, g2r)
    return out


def kernel(country_logits, region_logits, geocell_logits, pred_unit_xyz,
           uncertainty, embedding, positive_embedding, local_offset,
           base_centroid, geocell_probs, unit_xyz, geocell_centroids,
           country_id, region_id, geocell_id, geocell_to_country,
           geocell_to_region):
    B = country_logits.shape[0]
    out = _run(
        country_logits, region_logits, geocell_logits, geocell_probs,
        embedding, positive_embedding.T, positive_embedding,
        unit_xyz, pred_unit_xyz,
        local_offset, base_centroid, uncertainty, geocell_centroids.T,
        country_id.astype(jnp.int32).reshape(B, 1),
        region_id.astype(jnp.int32).reshape(B, 1),
        geocell_id.astype(jnp.int32).reshape(B, 1),
        geocell_to_country.astype(jnp.int32).reshape(-1, 1),
        geocell_to_region.astype(jnp.int32).reshape(-1, 1),
        BR=B // 16)
    return tuple(out[k] for k in range(8))
